# speedup vs baseline: 1.1966x; 1.1966x over previous
import jax
import jax.numpy as jnp
from jax.experimental import pallas as pl
from jax.experimental.pallas import tpu as pltpu

_B = 16
_C = 128
_L = 2048
_D2 = 256
_TK = 3
_N = _B * _L
_ROWS = _L
_EPS = 1e-5


def _xstats_kernel(x_ref, s_ref, q_ref):
    b = pl.program_id(0)
    xb = x_ref[0]
    s = jnp.sum(xb, axis=1, keepdims=True)
    q = jnp.sum(xb * xb, axis=1, keepdims=True)

    @pl.when(b == 0)
    def _init():
        s_ref[...] = s
        q_ref[...] = q

    @pl.when(b > 0)
    def _acc():
        s_ref[...] = s_ref[...] + s
        q_ref[...] = q_ref[...] + q


def _front_kernel(x_ref, xf_ref, scc_ref, shc_ref, scr_ref, shr_ref,
                  wc_ref, bsk_ref, w1t_ref, b1_ref,
                  res_ref, y1_ref, s1_ref, q1_ref):
    b = pl.program_id(0)
    xn = x_ref[0] * scc_ref[...] + shc_ref[...]
    z = jnp.zeros((_C, 1), jnp.float32)
    xm1 = jnp.concatenate([z, xn[:, :-1]], axis=1)
    xp1 = jnp.concatenate([xn[:, 1:], z], axis=1)
    xcat = jnp.concatenate([xm1, xn, xp1], axis=0)
    r = jnp.dot(wc_ref[...], xcat, preferred_element_type=jnp.float32)
    res_ref[0] = jnp.maximum(r + bsk_ref[...], 0.0)
    hf = xf_ref[...] * scr_ref[...] + shr_ref[...]
    y1 = jnp.dot(hf, w1t_ref[...], preferred_element_type=jnp.float32)
    y1 = y1 + b1_ref[...]
    y1_ref[...] = y1
    s = jnp.sum(y1, axis=0, keepdims=True)
    q = jnp.sum(y1 * y1, axis=0, keepdims=True)

    @pl.when(b == 0)
    def _init():
        s1_ref[...] = s
        q1_ref[...] = q

    @pl.when(b > 0)
    def _acc():
        s1_ref[...] = s1_ref[...] + s
        q1_ref[...] = q1_ref[...] + q


def _mid_kernel(y_ref, sc_ref, sh_ref, wt_ref, bias_ref,
                o_ref, s_ref, q_ref):
    i = pl.program_id(0)
    zz = jnp.maximum(y_ref[...] * sc_ref[...] + sh_ref[...], 0.0)
    y2 = jnp.dot(zz, wt_ref[...], preferred_element_type=jnp.float32)
    y2 = y2 + bias_ref[...]
    o_ref[...] = y2
    s = jnp.sum(y2, axis=0, keepdims=True)
    q = jnp.sum(y2 * y2, axis=0, keepdims=True)

    @pl.when(i == 0)
    def _init():
        s_ref[...] = s
        q_ref[...] = q

    @pl.when(i > 0)
    def _acc():
        s_ref[...] = s_ref[...] + s
        q_ref[...] = q_ref[...] + q


def _tail_kernel(y_ref, sc_ref, sh_ref, resf_ref, o_ref):
    zz = jnp.maximum(y_ref[...] * sc_ref[...] + sh_ref[...], 0.0)
    o_ref[...] = resf_ref[...] + zz


def _finalize(s, q, gamma, beta, count):
    mu = s / count
    var = q / count - mu * mu
    sc = gamma.reshape(mu.shape) * jax.lax.rsqrt(var + _EPS)
    sh = beta.reshape(mu.shape) - mu * sc
    return sc, sh


def kernel(x, edge_index, train, gamma0, beta0, Wskip, bskip, W1, bias1,
           gamma1, beta1, W2, bias2, W3, bias3):
    del edge_index, train
    f32 = jnp.float32
    xf = x.reshape(_N, _C)

    s0, q0 = pl.pallas_call(
        _xstats_kernel,
        grid=(_B,),
        in_specs=[pl.BlockSpec((1, _C, _L), lambda b: (b, 0, 0))],
        out_specs=[pl.BlockSpec((_C, 1), lambda b: (0, 0)),
                   pl.BlockSpec((_C, 1), lambda b: (0, 0))],
        out_shape=[jax.ShapeDtypeStruct((_C, 1), f32),
                   jax.ShapeDtypeStruct((_C, 1), f32)],
    )(x)

    sc0, sh0 = _finalize(s0, q0, gamma0.reshape(_C, 1), beta0.reshape(_C, 1),
                         float(_B * _L))
    sc0r = jnp.repeat(sc0.reshape(_C), _L // _C).reshape(_ROWS, 1)
    sh0r = jnp.repeat(sh0.reshape(_C), _L // _C).reshape(_ROWS, 1)

    wc = jnp.concatenate([Wskip[:, :, 0], Wskip[:, :, 1], Wskip[:, :, 2]],
                         axis=1)
    bsk = bskip.reshape(_D2, 1)
    w1t = W1.T
    b1 = bias1.reshape(1, _D2)

    res, y1, s1, q1 = pl.pallas_call(
        _front_kernel,
        grid=(_B,),
        in_specs=[
            pl.BlockSpec((1, _C, _L), lambda b: (b, 0, 0)),
            pl.BlockSpec((_ROWS, _C), lambda b: (b, 0)),
            pl.BlockSpec((_C, 1), lambda b: (0, 0)),
            pl.BlockSpec((_C, 1), lambda b: (0, 0)),
            pl.BlockSpec((_ROWS, 1), lambda b: (0, 0)),
            pl.BlockSpec((_ROWS, 1), lambda b: (0, 0)),
            pl.BlockSpec((_D2, _TK * _C), lambda b: (0, 0)),
            pl.BlockSpec((_D2, 1), lambda b: (0, 0)),
            pl.BlockSpec((_C, _D2), lambda b: (0, 0)),
            pl.BlockSpec((1, _D2), lambda b: (0, 0)),
        ],
        out_specs=[
            pl.BlockSpec((1, _D2, _L), lambda b: (b, 0, 0)),
            pl.BlockSpec((_ROWS, _D2), lambda b: (b, 0)),
            pl.BlockSpec((1, _D2), lambda b: (0, 0)),
            pl.BlockSpec((1, _D2), lambda b: (0, 0)),
        ],
        out_shape=[
            jax.ShapeDtypeStruct((_B, _D2, _L), f32),
            jax.ShapeDtypeStruct((_N, _D2), f32),
            jax.ShapeDtypeStruct((1, _D2), f32),
            jax.ShapeDtypeStruct((1, _D2), f32),
        ],
    )(x, xf, sc0, sh0, sc0r, sh0r, wc, bsk, w1t, b1)

    def mid(y, s, q, wt, bias):
        sc, sh = _finalize(s, q, gamma1.reshape(1, _D2),
                           beta1.reshape(1, _D2), float(_N))
        return pl.pallas_call(
            _mid_kernel,
            grid=(_B,),
            in_specs=[
                pl.BlockSpec((_ROWS, _D2), lambda b: (b, 0)),
                pl.BlockSpec((1, _D2), lambda b: (0, 0)),
                pl.BlockSpec((1, _D2), lambda b: (0, 0)),
                pl.BlockSpec((_D2, _D2), lambda b: (0, 0)),
                pl.BlockSpec((1, _D2), lambda b: (0, 0)),
            ],
            out_specs=[
                pl.BlockSpec((_ROWS, _D2), lambda b: (b, 0)),
                pl.BlockSpec((1, _D2), lambda b: (0, 0)),
                pl.BlockSpec((1, _D2), lambda b: (0, 0)),
            ],
            out_shape=[
                jax.ShapeDtypeStruct((_N, _D2), f32),
                jax.ShapeDtypeStruct((1, _D2), f32),
                jax.ShapeDtypeStruct((1, _D2), f32),
            ],
        )(y, sc, sh, wt, bias)

    y2, s2, q2 = mid(y1, s1, q1, W2.T, bias2.reshape(1, _D2))
    y3, s3, q3 = mid(y2, s2, q2, W3.T, bias3.reshape(1, _D2))

    sc3, sh3 = _finalize(s3, q3, gamma1.reshape(1, _D2),
                         beta1.reshape(1, _D2), float(_N))
    resf = res.reshape(_N, _D2)
    outf = pl.pallas_call(
        _tail_kernel,
        grid=(_B,),
        in_specs=[
            pl.BlockSpec((_ROWS, _D2), lambda b: (b, 0)),
            pl.BlockSpec((1, _D2), lambda b: (0, 0)),
            pl.BlockSpec((1, _D2), lambda b: (0, 0)),
            pl.BlockSpec((_ROWS, _D2), lambda b: (b, 0)),
        ],
        out_specs=pl.BlockSpec((_ROWS, _D2), lambda b: (b, 0)),
        out_shape=jax.ShapeDtypeStruct((_N, _D2), f32),
    )(y3, sc3, sh3, resf)

    return outf.reshape(_B, _D2, _L)


# speedup vs baseline: 1.2041x; 1.0063x over previous
import jax
import jax.numpy as jnp
from jax.experimental import pallas as pl
from jax.experimental.pallas import tpu as pltpu

_B = 16
_C = 128
_L = 2048
_D2 = 256
_TK = 3
_N = _B * _L
_ROWS = _L
_EPS = 1e-5


def _xstats_kernel(x_ref, s_ref, q_ref):
    b = pl.program_id(0)
    xb = x_ref[0]
    s = jnp.sum(xb, axis=1, keepdims=True)
    q = jnp.sum(xb * xb, axis=1, keepdims=True)

    @pl.when(b == 0)
    def _init():
        s_ref[...] = s
        q_ref[...] = q

    @pl.when(b > 0)
    def _acc():
        s_ref[...] = s_ref[...] + s
        q_ref[...] = q_ref[...] + q


def _front_kernel(x_ref, xf_ref, scc_ref, shc_ref, scr_ref, shr_ref,
                  wc_ref, bsk_ref, w1t_ref, b1_ref,
                  res_ref, y1_ref, s1_ref, q1_ref):
    b = pl.program_id(0)
    xn = x_ref[0] * scc_ref[...] + shc_ref[...]
    z = jnp.zeros((_C, 1), jnp.float32)
    xm1 = jnp.concatenate([z, xn[:, :-1]], axis=1)
    xp1 = jnp.concatenate([xn[:, 1:], z], axis=1)
    xcat = jnp.concatenate([xm1, xn, xp1], axis=0)
    r = jnp.dot(wc_ref[...], xcat.astype(jnp.bfloat16),
                preferred_element_type=jnp.float32)
    res_ref[0] = jnp.maximum(r + bsk_ref[...], 0.0)
    hf = xf_ref[...] * scr_ref[...] + shr_ref[...]
    y1 = jnp.dot(hf.astype(jnp.bfloat16), w1t_ref[...],
                 preferred_element_type=jnp.float32)
    y1 = y1 + b1_ref[...]
    y1_ref[...] = y1
    s = jnp.sum(y1, axis=0, keepdims=True)
    q = jnp.sum(y1 * y1, axis=0, keepdims=True)

    @pl.when(b == 0)
    def _init():
        s1_ref[...] = s
        q1_ref[...] = q

    @pl.when(b > 0)
    def _acc():
        s1_ref[...] = s1_ref[...] + s
        q1_ref[...] = q1_ref[...] + q


def _mid_kernel(y_ref, sc_ref, sh_ref, wt_ref, bias_ref,
                o_ref, s_ref, q_ref):
    i = pl.program_id(0)
    zz = jnp.maximum(y_ref[...] * sc_ref[...] + sh_ref[...], 0.0)
    y2 = jnp.dot(zz.astype(jnp.bfloat16), wt_ref[...],
                 preferred_element_type=jnp.float32)
    y2 = y2 + bias_ref[...]
    o_ref[...] = y2
    s = jnp.sum(y2, axis=0, keepdims=True)
    q = jnp.sum(y2 * y2, axis=0, keepdims=True)

    @pl.when(i == 0)
    def _init():
        s_ref[...] = s
        q_ref[...] = q

    @pl.when(i > 0)
    def _acc():
        s_ref[...] = s_ref[...] + s
        q_ref[...] = q_ref[...] + q


def _tail_kernel(y_ref, sc_ref, sh_ref, resf_ref, o_ref):
    zz = jnp.maximum(y_ref[...] * sc_ref[...] + sh_ref[...], 0.0)
    o_ref[...] = resf_ref[...] + zz


def _finalize(s, q, gamma, beta, count):
    mu = s / count
    var = q / count - mu * mu
    sc = gamma.reshape(mu.shape) * jax.lax.rsqrt(var + _EPS)
    sh = beta.reshape(mu.shape) - mu * sc
    return sc, sh


def kernel(x, edge_index, train, gamma0, beta0, Wskip, bskip, W1, bias1,
           gamma1, beta1, W2, bias2, W3, bias3):
    del edge_index, train
    f32 = jnp.float32
    xf = x.reshape(_N, _C)

    s0, q0 = pl.pallas_call(
        _xstats_kernel,
        grid=(_B,),
        in_specs=[pl.BlockSpec((1, _C, _L), lambda b: (b, 0, 0))],
        out_specs=[pl.BlockSpec((_C, 1), lambda b: (0, 0)),
                   pl.BlockSpec((_C, 1), lambda b: (0, 0))],
        out_shape=[jax.ShapeDtypeStruct((_C, 1), f32),
                   jax.ShapeDtypeStruct((_C, 1), f32)],
    )(x)

    sc0, sh0 = _finalize(s0, q0, gamma0.reshape(_C, 1), beta0.reshape(_C, 1),
                         float(_B * _L))
    sc0r = jnp.repeat(sc0.reshape(_C), _L // _C).reshape(_ROWS, 1)
    sh0r = jnp.repeat(sh0.reshape(_C), _L // _C).reshape(_ROWS, 1)

    wc = jnp.concatenate([Wskip[:, :, 0], Wskip[:, :, 1], Wskip[:, :, 2]],
                         axis=1).astype(jnp.bfloat16)
    bsk = bskip.reshape(_D2, 1)
    w1t = W1.T.astype(jnp.bfloat16)
    b1 = bias1.reshape(1, _D2)

    res, y1, s1, q1 = pl.pallas_call(
        _front_kernel,
        grid=(_B,),
        in_specs=[
            pl.BlockSpec((1, _C, _L), lambda b: (b, 0, 0)),
            pl.BlockSpec((_ROWS, _C), lambda b: (b, 0)),
            pl.BlockSpec((_C, 1), lambda b: (0, 0)),
            pl.BlockSpec((_C, 1), lambda b: (0, 0)),
            pl.BlockSpec((_ROWS, 1), lambda b: (0, 0)),
            pl.BlockSpec((_ROWS, 1), lambda b: (0, 0)),
            pl.BlockSpec((_D2, _TK * _C), lambda b: (0, 0)),
            pl.BlockSpec((_D2, 1), lambda b: (0, 0)),
            pl.BlockSpec((_C, _D2), lambda b: (0, 0)),
            pl.BlockSpec((1, _D2), lambda b: (0, 0)),
        ],
        out_specs=[
            pl.BlockSpec((1, _D2, _L), lambda b: (b, 0, 0)),
            pl.BlockSpec((_ROWS, _D2), lambda b: (b, 0)),
            pl.BlockSpec((1, _D2), lambda b: (0, 0)),
            pl.BlockSpec((1, _D2), lambda b: (0, 0)),
        ],
        out_shape=[
            jax.ShapeDtypeStruct((_B, _D2, _L), f32),
            jax.ShapeDtypeStruct((_N, _D2), f32),
            jax.ShapeDtypeStruct((1, _D2), f32),
            jax.ShapeDtypeStruct((1, _D2), f32),
        ],
    )(x, xf, sc0, sh0, sc0r, sh0r, wc, bsk, w1t, b1)

    def mid(y, s, q, wt, bias):
        sc, sh = _finalize(s, q, gamma1.reshape(1, _D2),
                           beta1.reshape(1, _D2), float(_N))
        return pl.pallas_call(
            _mid_kernel,
            grid=(_B,),
            in_specs=[
                pl.BlockSpec((_ROWS, _D2), lambda b: (b, 0)),
                pl.BlockSpec((1, _D2), lambda b: (0, 0)),
                pl.BlockSpec((1, _D2), lambda b: (0, 0)),
                pl.BlockSpec((_D2, _D2), lambda b: (0, 0)),
                pl.BlockSpec((1, _D2), lambda b: (0, 0)),
            ],
            out_specs=[
                pl.BlockSpec((_ROWS, _D2), lambda b: (b, 0)),
                pl.BlockSpec((1, _D2), lambda b: (0, 0)),
                pl.BlockSpec((1, _D2), lambda b: (0, 0)),
            ],
            out_shape=[
                jax.ShapeDtypeStruct((_N, _D2), f32),
                jax.ShapeDtypeStruct((1, _D2), f32),
                jax.ShapeDtypeStruct((1, _D2), f32),
            ],
        )(y, sc, sh, wt, bias)

    y2, s2, q2 = mid(y1, s1, q1, W2.T.astype(jnp.bfloat16),
                     bias2.reshape(1, _D2))
    y3, s3, q3 = mid(y2, s2, q2, W3.T.astype(jnp.bfloat16),
                     bias3.reshape(1, _D2))

    sc3, sh3 = _finalize(s3, q3, gamma1.reshape(1, _D2),
                         beta1.reshape(1, _D2), float(_N))
    resf = res.reshape(_N, _D2)
    outf = pl.pallas_call(
        _tail_kernel,
        grid=(_B,),
        in_specs=[
            pl.BlockSpec((_ROWS, _D2), lambda b: (b, 0)),
            pl.BlockSpec((1, _D2), lambda b: (0, 0)),
            pl.BlockSpec((1, _D2), lambda b: (0, 0)),
            pl.BlockSpec((_ROWS, _D2), lambda b: (b, 0)),
        ],
        out_specs=pl.BlockSpec((_ROWS, _D2), lambda b: (b, 0)),
        out_shape=jax.ShapeDtypeStruct((_N, _D2), f32),
    )(y3, sc3, sh3, resf)

    return outf.reshape(_B, _D2, _L)


# speedup vs baseline: 1.9600x; 1.6278x over previous
import jax
import jax.numpy as jnp
from jax.experimental import pallas as pl
from jax.experimental.pallas import tpu as pltpu

_B = 16
_C = 128
_L = 2048
_D2 = 256
_TK = 3
_N = _B * _L
_ROWS = _L
_EPS = 1e-5


def _xstats_kernel(x_ref, s_ref, q_ref):
    b = pl.program_id(0)
    xb = x_ref[0]
    s = jnp.sum(xb, axis=1, keepdims=True)
    q = jnp.sum(xb * xb, axis=1, keepdims=True)

    @pl.when(b == 0)
    def _init():
        s_ref[...] = s
        q_ref[...] = q

    @pl.when(b > 0)
    def _acc():
        s_ref[...] = s_ref[...] + s
        q_ref[...] = q_ref[...] + q


def _front_kernel(x_ref, scc_ref, shc_ref,
                  wc_ref, bsk_ref, w1t_ref, b1_ref,
                  res_ref, y1_ref, s1_ref, q1_ref):
    b = pl.program_id(0)
    xn = x_ref[0] * scc_ref[...] + shc_ref[...]
    z = jnp.zeros((_C, 1), jnp.float32)
    xm1 = jnp.concatenate([z, xn[:, :-1]], axis=1)
    xp1 = jnp.concatenate([xn[:, 1:], z], axis=1)
    xcat = jnp.concatenate([xm1, xn, xp1], axis=0)
    r = jnp.dot(wc_ref[...], xcat.astype(jnp.bfloat16),
                preferred_element_type=jnp.float32)
    res_ref[0] = jnp.maximum(r + bsk_ref[...], 0.0)
    hf = jnp.reshape(xn, (_ROWS, _C))
    y1 = jnp.dot(hf.astype(jnp.bfloat16), w1t_ref[...],
                 preferred_element_type=jnp.float32)
    y1 = y1 + b1_ref[...]
    y1_ref[...] = y1
    s = jnp.sum(y1, axis=0, keepdims=True)
    q = jnp.sum(y1 * y1, axis=0, keepdims=True)

    @pl.when(b == 0)
    def _init():
        s1_ref[...] = s
        q1_ref[...] = q

    @pl.when(b > 0)
    def _acc():
        s1_ref[...] = s1_ref[...] + s
        q1_ref[...] = q1_ref[...] + q


def _mid_kernel(y_ref, sc_ref, sh_ref, wt_ref, bias_ref,
                o_ref, s_ref, q_ref):
    i = pl.program_id(0)
    zz = jnp.maximum(y_ref[...] * sc_ref[...] + sh_ref[...], 0.0)
    y2 = jnp.dot(zz.astype(jnp.bfloat16), wt_ref[...],
                 preferred_element_type=jnp.float32)
    y2 = y2 + bias_ref[...]
    o_ref[...] = y2
    s = jnp.sum(y2, axis=0, keepdims=True)
    q = jnp.sum(y2 * y2, axis=0, keepdims=True)

    @pl.when(i == 0)
    def _init():
        s_ref[...] = s
        q_ref[...] = q

    @pl.when(i > 0)
    def _acc():
        s_ref[...] = s_ref[...] + s
        q_ref[...] = q_ref[...] + q


def _tail_kernel(y_ref, sc_ref, sh_ref, res_ref, o_ref):
    zz = jnp.maximum(y_ref[...] * sc_ref[...] + sh_ref[...], 0.0)
    o_ref[0] = res_ref[0] + jnp.reshape(zz, (_D2, _L))


def _finalize(s, q, gamma, beta, count):
    mu = s / count
    var = q / count - mu * mu
    sc = gamma.reshape(mu.shape) * jax.lax.rsqrt(var + _EPS)
    sh = beta.reshape(mu.shape) - mu * sc
    return sc, sh


def kernel(x, edge_index, train, gamma0, beta0, Wskip, bskip, W1, bias1,
           gamma1, beta1, W2, bias2, W3, bias3):
    del edge_index, train
    f32 = jnp.float32

    s0, q0 = pl.pallas_call(
        _xstats_kernel,
        grid=(_B,),
        in_specs=[pl.BlockSpec((1, _C, _L), lambda b: (b, 0, 0))],
        out_specs=[pl.BlockSpec((_C, 1), lambda b: (0, 0)),
                   pl.BlockSpec((_C, 1), lambda b: (0, 0))],
        out_shape=[jax.ShapeDtypeStruct((_C, 1), f32),
                   jax.ShapeDtypeStruct((_C, 1), f32)],
    )(x)

    sc0, sh0 = _finalize(s0, q0, gamma0.reshape(_C, 1), beta0.reshape(_C, 1),
                         float(_B * _L))

    wc = jnp.concatenate([Wskip[:, :, 0], Wskip[:, :, 1], Wskip[:, :, 2]],
                         axis=1).astype(jnp.bfloat16)
    bsk = bskip.reshape(_D2, 1)
    w1t = W1.T.astype(jnp.bfloat16)
    b1 = bias1.reshape(1, _D2)

    res, y1, s1, q1 = pl.pallas_call(
        _front_kernel,
        grid=(_B,),
        in_specs=[
            pl.BlockSpec((1, _C, _L), lambda b: (b, 0, 0)),
            pl.BlockSpec((_C, 1), lambda b: (0, 0)),
            pl.BlockSpec((_C, 1), lambda b: (0, 0)),
            pl.BlockSpec((_D2, _TK * _C), lambda b: (0, 0)),
            pl.BlockSpec((_D2, 1), lambda b: (0, 0)),
            pl.BlockSpec((_C, _D2), lambda b: (0, 0)),
            pl.BlockSpec((1, _D2), lambda b: (0, 0)),
        ],
        out_specs=[
            pl.BlockSpec((1, _D2, _L), lambda b: (b, 0, 0)),
            pl.BlockSpec((_ROWS, _D2), lambda b: (b, 0)),
            pl.BlockSpec((1, _D2), lambda b: (0, 0)),
            pl.BlockSpec((1, _D2), lambda b: (0, 0)),
        ],
        out_shape=[
            jax.ShapeDtypeStruct((_B, _D2, _L), f32),
            jax.ShapeDtypeStruct((_N, _D2), f32),
            jax.ShapeDtypeStruct((1, _D2), f32),
            jax.ShapeDtypeStruct((1, _D2), f32),
        ],
    )(x, sc0, sh0, wc, bsk, w1t, b1)

    def mid(y, s, q, wt, bias):
        sc, sh = _finalize(s, q, gamma1.reshape(1, _D2),
                           beta1.reshape(1, _D2), float(_N))
        return pl.pallas_call(
            _mid_kernel,
            grid=(_B,),
            in_specs=[
                pl.BlockSpec((_ROWS, _D2), lambda b: (b, 0)),
                pl.BlockSpec((1, _D2), lambda b: (0, 0)),
                pl.BlockSpec((1, _D2), lambda b: (0, 0)),
                pl.BlockSpec((_D2, _D2), lambda b: (0, 0)),
                pl.BlockSpec((1, _D2), lambda b: (0, 0)),
            ],
            out_specs=[
                pl.BlockSpec((_ROWS, _D2), lambda b: (b, 0)),
                pl.BlockSpec((1, _D2), lambda b: (0, 0)),
                pl.BlockSpec((1, _D2), lambda b: (0, 0)),
            ],
            out_shape=[
                jax.ShapeDtypeStruct((_N, _D2), f32),
                jax.ShapeDtypeStruct((1, _D2), f32),
                jax.ShapeDtypeStruct((1, _D2), f32),
            ],
        )(y, sc, sh, wt, bias)

    y2, s2, q2 = mid(y1, s1, q1, W2.T.astype(jnp.bfloat16),
                     bias2.reshape(1, _D2))
    y3, s3, q3 = mid(y2, s2, q2, W3.T.astype(jnp.bfloat16),
                     bias3.reshape(1, _D2))

    sc3, sh3 = _finalize(s3, q3, gamma1.reshape(1, _D2),
                         beta1.reshape(1, _D2), float(_N))
    out = pl.pallas_call(
        _tail_kernel,
        grid=(_B,),
        in_specs=[
            pl.BlockSpec((_ROWS, _D2), lambda b: (b, 0)),
            pl.BlockSpec((1, _D2), lambda b: (0, 0)),
            pl.BlockSpec((1, _D2), lambda b: (0, 0)),
            pl.BlockSpec((1, _D2, _L), lambda b: (b, 0, 0)),
        ],
        out_specs=pl.BlockSpec((1, _D2, _L), lambda b: (b, 0, 0)),
        out_shape=jax.ShapeDtypeStruct((_B, _D2, _L), f32),
    )(y3, sc3, sh3, res)

    return out


# speedup vs baseline: 2.3213x; 1.1843x over previous
import jax
import jax.numpy as jnp
from jax.experimental import pallas as pl
from jax.experimental.pallas import tpu as pltpu

_B = 16
_C = 128
_L = 2048
_D2 = 256
_TK = 3
_N = _B * _L
_ROWS = _L
_EPS = 1e-5


def _xstats_kernel(x_ref, s_ref, q_ref):
    b = pl.program_id(0)
    xb = x_ref[...]
    s = jnp.sum(xb, axis=(0, 2))[:, None]
    q = jnp.sum(xb * xb, axis=(0, 2))[:, None]

    @pl.when(b == 0)
    def _init():
        s_ref[...] = s
        q_ref[...] = q

    @pl.when(b > 0)
    def _acc():
        s_ref[...] = s_ref[...] + s
        q_ref[...] = q_ref[...] + q


def _front_kernel(x_ref, scc_ref, shc_ref,
                  wc_ref, bsk_ref, w1t_ref, b1_ref,
                  res_ref, y1_ref, s1_ref, q1_ref):
    b = pl.program_id(0)
    xn = x_ref[0] * scc_ref[...] + shc_ref[...]
    z = jnp.zeros((_C, 1), jnp.float32)
    xm1 = jnp.concatenate([z, xn[:, :-1]], axis=1)
    xp1 = jnp.concatenate([xn[:, 1:], z], axis=1)
    xcat = jnp.concatenate([xm1, xn, xp1], axis=0)
    r = jnp.dot(wc_ref[...], xcat.astype(jnp.bfloat16),
                preferred_element_type=jnp.float32)
    res_ref[0] = jnp.maximum(r + bsk_ref[...], 0.0)
    hf = jnp.reshape(xn, (_ROWS, _C))
    y1 = jnp.dot(hf.astype(jnp.bfloat16), w1t_ref[...],
                 preferred_element_type=jnp.float32)
    y1 = y1 + b1_ref[...]
    y1_ref[...] = y1.astype(jnp.bfloat16)
    s = jnp.sum(y1, axis=0, keepdims=True)
    q = jnp.sum(y1 * y1, axis=0, keepdims=True)

    @pl.when(b == 0)
    def _init():
        s1_ref[...] = s
        q1_ref[...] = q

    @pl.when(b > 0)
    def _acc():
        s1_ref[...] = s1_ref[...] + s
        q1_ref[...] = q1_ref[...] + q


def _mid_kernel(y_ref, sc_ref, sh_ref, wt_ref, bias_ref,
                o_ref, s_ref, q_ref):
    i = pl.program_id(0)
    yv = y_ref[...].astype(jnp.float32)
    zz = jnp.maximum(yv * sc_ref[...] + sh_ref[...], 0.0)
    y2 = jnp.dot(zz.astype(jnp.bfloat16), wt_ref[...],
                 preferred_element_type=jnp.float32)
    y2 = y2 + bias_ref[...]
    o_ref[...] = y2.astype(jnp.bfloat16)
    s = jnp.sum(y2, axis=0, keepdims=True)
    q = jnp.sum(y2 * y2, axis=0, keepdims=True)

    @pl.when(i == 0)
    def _init():
        s_ref[...] = s
        q_ref[...] = q

    @pl.when(i > 0)
    def _acc():
        s_ref[...] = s_ref[...] + s
        q_ref[...] = q_ref[...] + q


def _tail_kernel(y_ref, sc_ref, sh_ref, res_ref, o_ref):
    yv = y_ref[...].astype(jnp.float32)
    zz = jnp.maximum(yv * sc_ref[...] + sh_ref[...], 0.0)
    o_ref[0] = res_ref[0] + jnp.reshape(zz, (_D2, _L))


def _finalize(s, q, gamma, beta, count):
    mu = s / count
    var = q / count - mu * mu
    sc = gamma.reshape(mu.shape) * jax.lax.rsqrt(var + _EPS)
    sh = beta.reshape(mu.shape) - mu * sc
    return sc, sh


def kernel(x, edge_index, train, gamma0, beta0, Wskip, bskip, W1, bias1,
           gamma1, beta1, W2, bias2, W3, bias3):
    del edge_index, train
    f32 = jnp.float32

    s0, q0 = pl.pallas_call(
        _xstats_kernel,
        grid=(_B // 4,),
        in_specs=[pl.BlockSpec((4, _C, _L), lambda b: (b, 0, 0))],
        out_specs=[pl.BlockSpec((_C, 1), lambda b: (0, 0)),
                   pl.BlockSpec((_C, 1), lambda b: (0, 0))],
        out_shape=[jax.ShapeDtypeStruct((_C, 1), f32),
                   jax.ShapeDtypeStruct((_C, 1), f32)],
    )(x)

    sc0, sh0 = _finalize(s0, q0, gamma0.reshape(_C, 1), beta0.reshape(_C, 1),
                         float(_B * _L))

    wc = jnp.concatenate([Wskip[:, :, 0], Wskip[:, :, 1], Wskip[:, :, 2]],
                         axis=1).astype(jnp.bfloat16)
    bsk = bskip.reshape(_D2, 1)
    w1t = W1.T.astype(jnp.bfloat16)
    b1 = bias1.reshape(1, _D2)

    res, y1, s1, q1 = pl.pallas_call(
        _front_kernel,
        grid=(_B,),
        in_specs=[
            pl.BlockSpec((1, _C, _L), lambda b: (b, 0, 0)),
            pl.BlockSpec((_C, 1), lambda b: (0, 0)),
            pl.BlockSpec((_C, 1), lambda b: (0, 0)),
            pl.BlockSpec((_D2, _TK * _C), lambda b: (0, 0)),
            pl.BlockSpec((_D2, 1), lambda b: (0, 0)),
            pl.BlockSpec((_C, _D2), lambda b: (0, 0)),
            pl.BlockSpec((1, _D2), lambda b: (0, 0)),
        ],
        out_specs=[
            pl.BlockSpec((1, _D2, _L), lambda b: (b, 0, 0)),
            pl.BlockSpec((_ROWS, _D2), lambda b: (b, 0)),
            pl.BlockSpec((1, _D2), lambda b: (0, 0)),
            pl.BlockSpec((1, _D2), lambda b: (0, 0)),
        ],
        out_shape=[
            jax.ShapeDtypeStruct((_B, _D2, _L), f32),
            jax.ShapeDtypeStruct((_N, _D2), jnp.bfloat16),
            jax.ShapeDtypeStruct((1, _D2), f32),
            jax.ShapeDtypeStruct((1, _D2), f32),
        ],
    )(x, sc0, sh0, wc, bsk, w1t, b1)

    def mid(y, s, q, wt, bias):
        sc, sh = _finalize(s, q, gamma1.reshape(1, _D2),
                           beta1.reshape(1, _D2), float(_N))
        return pl.pallas_call(
            _mid_kernel,
            grid=(_B,),
            in_specs=[
                pl.BlockSpec((_ROWS, _D2), lambda b: (b, 0)),
                pl.BlockSpec((1, _D2), lambda b: (0, 0)),
                pl.BlockSpec((1, _D2), lambda b: (0, 0)),
                pl.BlockSpec((_D2, _D2), lambda b: (0, 0)),
                pl.BlockSpec((1, _D2), lambda b: (0, 0)),
            ],
            out_specs=[
                pl.BlockSpec((_ROWS, _D2), lambda b: (b, 0)),
                pl.BlockSpec((1, _D2), lambda b: (0, 0)),
                pl.BlockSpec((1, _D2), lambda b: (0, 0)),
            ],
            out_shape=[
                jax.ShapeDtypeStruct((_N, _D2), jnp.bfloat16),
                jax.ShapeDtypeStruct((1, _D2), f32),
                jax.ShapeDtypeStruct((1, _D2), f32),
            ],
        )(y, sc, sh, wt, bias)

    y2, s2, q2 = mid(y1, s1, q1, W2.T.astype(jnp.bfloat16),
                     bias2.reshape(1, _D2))
    y3, s3, q3 = mid(y2, s2, q2, W3.T.astype(jnp.bfloat16),
                     bias3.reshape(1, _D2))

    sc3, sh3 = _finalize(s3, q3, gamma1.reshape(1, _D2),
                         beta1.reshape(1, _D2), float(_N))
    out = pl.pallas_call(
        _tail_kernel,
        grid=(_B,),
        in_specs=[
            pl.BlockSpec((_ROWS, _D2), lambda b: (b, 0)),
            pl.BlockSpec((1, _D2), lambda b: (0, 0)),
            pl.BlockSpec((1, _D2), lambda b: (0, 0)),
            pl.BlockSpec((1, _D2, _L), lambda b: (b, 0, 0)),
        ],
        out_specs=pl.BlockSpec((1, _D2, _L), lambda b: (b, 0, 0)),
        out_shape=jax.ShapeDtypeStruct((_B, _D2, _L), f32),
    )(y3, sc3, sh3, res)

    return out


# speedup vs baseline: 2.6975x; 1.1621x over previous
import jax
import jax.numpy as jnp
from jax.experimental import pallas as pl
from jax.experimental.pallas import tpu as pltpu

_B = 16
_C = 128
_L = 2048
_D2 = 256
_TK = 3
_N = _B * _L
_ROWS = _L
_EPS = 1e-5
_FB = 2
_MB = 4


def _xstats_kernel(x_ref, s_ref, q_ref):
    b = pl.program_id(0)
    xb = x_ref[...]
    s = jnp.sum(xb, axis=(0, 2))[:, None]
    q = jnp.sum(xb * xb, axis=(0, 2))[:, None]

    @pl.when(b == 0)
    def _init():
        s_ref[...] = s
        q_ref[...] = q

    @pl.when(b > 0)
    def _acc():
        s_ref[...] = s_ref[...] + s
        q_ref[...] = q_ref[...] + q


def _front_kernel(x_ref, scc_ref, shc_ref,
                  wc_ref, bsk_ref, w1t_ref, b1_ref,
                  res_ref, y1_ref, s1_ref, q1_ref):
    b = pl.program_id(0)
    s = jnp.zeros((1, _D2), jnp.float32)
    q = jnp.zeros((1, _D2), jnp.float32)
    for t in range(_FB):
        xn = x_ref[t] * scc_ref[...] + shc_ref[...]
        z = jnp.zeros((_C, 1), jnp.float32)
        xm1 = jnp.concatenate([z, xn[:, :-1]], axis=1)
        xp1 = jnp.concatenate([xn[:, 1:], z], axis=1)
        xcat = jnp.concatenate([xm1, xn, xp1], axis=0)
        r = jnp.dot(wc_ref[...], xcat.astype(jnp.bfloat16),
                    preferred_element_type=jnp.float32)
        res_ref[t] = jnp.maximum(r + bsk_ref[...], 0.0)
        hf = jnp.reshape(xn, (_ROWS, _C))
        y1 = jnp.dot(hf.astype(jnp.bfloat16), w1t_ref[...],
                     preferred_element_type=jnp.float32)
        y1 = y1 + b1_ref[...]
        y1_ref[t * _ROWS:(t + 1) * _ROWS, :] = y1.astype(jnp.bfloat16)
        s = s + jnp.sum(y1, axis=0, keepdims=True)
        q = q + jnp.sum(y1 * y1, axis=0, keepdims=True)

    @pl.when(b == 0)
    def _init():
        s1_ref[...] = s
        q1_ref[...] = q

    @pl.when(b > 0)
    def _acc():
        s1_ref[...] = s1_ref[...] + s
        q1_ref[...] = q1_ref[...] + q


def _mid_kernel(y_ref, sc_ref, sh_ref, wt_ref, bias_ref,
                o_ref, s_ref, q_ref):
    i = pl.program_id(0)
    yv = y_ref[...].astype(jnp.float32)
    zz = jnp.maximum(yv * sc_ref[...] + sh_ref[...], 0.0)
    y2 = jnp.dot(zz.astype(jnp.bfloat16), wt_ref[...],
                 preferred_element_type=jnp.float32)
    y2 = y2 + bias_ref[...]
    o_ref[...] = y2.astype(jnp.bfloat16)
    s = jnp.sum(y2, axis=0, keepdims=True)
    q = jnp.sum(y2 * y2, axis=0, keepdims=True)

    @pl.when(i == 0)
    def _init():
        s_ref[...] = s
        q_ref[...] = q

    @pl.when(i > 0)
    def _acc():
        s_ref[...] = s_ref[...] + s
        q_ref[...] = q_ref[...] + q


def _tail_kernel(y_ref, sc_ref, sh_ref, res_ref, o_ref):
    yv = y_ref[...].astype(jnp.float32)
    zz = jnp.maximum(yv * sc_ref[...] + sh_ref[...], 0.0)
    o_ref[...] = res_ref[...] + jnp.reshape(zz, (_MB, _D2, _L))


def _finalize(s, q, gamma, beta, count):
    mu = s / count
    var = q / count - mu * mu
    sc = gamma.reshape(mu.shape) * jax.lax.rsqrt(var + _EPS)
    sh = beta.reshape(mu.shape) - mu * sc
    return sc, sh


def kernel(x, edge_index, train, gamma0, beta0, Wskip, bskip, W1, bias1,
           gamma1, beta1, W2, bias2, W3, bias3):
    del edge_index, train
    f32 = jnp.float32

    s0, q0 = pl.pallas_call(
        _xstats_kernel,
        grid=(_B // 4,),
        in_specs=[pl.BlockSpec((4, _C, _L), lambda b: (b, 0, 0))],
        out_specs=[pl.BlockSpec((_C, 1), lambda b: (0, 0)),
                   pl.BlockSpec((_C, 1), lambda b: (0, 0))],
        out_shape=[jax.ShapeDtypeStruct((_C, 1), f32),
                   jax.ShapeDtypeStruct((_C, 1), f32)],
    )(x)

    sc0, sh0 = _finalize(s0, q0, gamma0.reshape(_C, 1), beta0.reshape(_C, 1),
                         float(_B * _L))

    wc = jnp.concatenate([Wskip[:, :, 0], Wskip[:, :, 1], Wskip[:, :, 2]],
                         axis=1).astype(jnp.bfloat16)
    bsk = bskip.reshape(_D2, 1)
    w1t = W1.T.astype(jnp.bfloat16)
    b1 = bias1.reshape(1, _D2)

    res, y1, s1, q1 = pl.pallas_call(
        _front_kernel,
        grid=(_B // _FB,),
        in_specs=[
            pl.BlockSpec((_FB, _C, _L), lambda b: (b, 0, 0)),
            pl.BlockSpec((_C, 1), lambda b: (0, 0)),
            pl.BlockSpec((_C, 1), lambda b: (0, 0)),
            pl.BlockSpec((_D2, _TK * _C), lambda b: (0, 0)),
            pl.BlockSpec((_D2, 1), lambda b: (0, 0)),
            pl.BlockSpec((_C, _D2), lambda b: (0, 0)),
            pl.BlockSpec((1, _D2), lambda b: (0, 0)),
        ],
        out_specs=[
            pl.BlockSpec((_FB, _D2, _L), lambda b: (b, 0, 0)),
            pl.BlockSpec((_FB * _ROWS, _D2), lambda b: (b, 0)),
            pl.BlockSpec((1, _D2), lambda b: (0, 0)),
            pl.BlockSpec((1, _D2), lambda b: (0, 0)),
        ],
        out_shape=[
            jax.ShapeDtypeStruct((_B, _D2, _L), f32),
            jax.ShapeDtypeStruct((_N, _D2), jnp.bfloat16),
            jax.ShapeDtypeStruct((1, _D2), f32),
            jax.ShapeDtypeStruct((1, _D2), f32),
        ],
    )(x, sc0, sh0, wc, bsk, w1t, b1)

    def mid(y, s, q, wt, bias):
        sc, sh = _finalize(s, q, gamma1.reshape(1, _D2),
                           beta1.reshape(1, _D2), float(_N))
        return pl.pallas_call(
            _mid_kernel,
            grid=(_B // _MB,),
            in_specs=[
                pl.BlockSpec((_MB * _ROWS, _D2), lambda b: (b, 0)),
                pl.BlockSpec((1, _D2), lambda b: (0, 0)),
                pl.BlockSpec((1, _D2), lambda b: (0, 0)),
                pl.BlockSpec((_D2, _D2), lambda b: (0, 0)),
                pl.BlockSpec((1, _D2), lambda b: (0, 0)),
            ],
            out_specs=[
                pl.BlockSpec((_MB * _ROWS, _D2), lambda b: (b, 0)),
                pl.BlockSpec((1, _D2), lambda b: (0, 0)),
                pl.BlockSpec((1, _D2), lambda b: (0, 0)),
            ],
            out_shape=[
                jax.ShapeDtypeStruct((_N, _D2), jnp.bfloat16),
                jax.ShapeDtypeStruct((1, _D2), f32),
                jax.ShapeDtypeStruct((1, _D2), f32),
            ],
        )(y, sc, sh, wt, bias)

    y2, s2, q2 = mid(y1, s1, q1, W2.T.astype(jnp.bfloat16),
                     bias2.reshape(1, _D2))
    y3, s3, q3 = mid(y2, s2, q2, W3.T.astype(jnp.bfloat16),
                     bias3.reshape(1, _D2))

    sc3, sh3 = _finalize(s3, q3, gamma1.reshape(1, _D2),
                         beta1.reshape(1, _D2), float(_N))
    out = pl.pallas_call(
        _tail_kernel,
        grid=(_B // _MB,),
        in_specs=[
            pl.BlockSpec((_MB * _ROWS, _D2), lambda b: (b, 0)),
            pl.BlockSpec((1, _D2), lambda b: (0, 0)),
            pl.BlockSpec((1, _D2), lambda b: (0, 0)),
            pl.BlockSpec((_MB, _D2, _L), lambda b: (b, 0, 0)),
        ],
        out_specs=pl.BlockSpec((_MB, _D2, _L), lambda b: (b, 0, 0)),
        out_shape=jax.ShapeDtypeStruct((_B, _D2, _L), f32),
    )(y3, sc3, sh3, res)

    return out


# speedup vs baseline: 2.8778x; 1.0668x over previous
import jax
import jax.numpy as jnp
from jax.experimental import pallas as pl
from jax.experimental.pallas import tpu as pltpu

_B = 16
_C = 128
_L = 2048
_D2 = 256
_TK = 3
_N = _B * _L
_ROWS = _L
_EPS = 1e-5
_FB = 4
_MB = 4
_TB = 2


def _xstats_kernel(x_ref, s_ref, q_ref):
    b = pl.program_id(0)
    xb = x_ref[...]
    s = jnp.sum(xb, axis=(0, 2))[:, None]
    q = jnp.sum(xb * xb, axis=(0, 2))[:, None]

    @pl.when(b == 0)
    def _init():
        s_ref[...] = s
        q_ref[...] = q

    @pl.when(b > 0)
    def _acc():
        s_ref[...] = s_ref[...] + s
        q_ref[...] = q_ref[...] + q


def _front_kernel(x_ref, scc_ref, shc_ref, w1t_ref, b1_ref,
                  y1_ref, s1_ref, q1_ref):
    b = pl.program_id(0)
    s = jnp.zeros((1, _D2), jnp.float32)
    q = jnp.zeros((1, _D2), jnp.float32)
    for t in range(_FB):
        xn = x_ref[t] * scc_ref[...] + shc_ref[...]
        hf = jnp.reshape(xn, (_ROWS, _C))
        y1 = jnp.dot(hf.astype(jnp.bfloat16), w1t_ref[...],
                     preferred_element_type=jnp.float32)
        y1 = y1 + b1_ref[...]
        y1_ref[t * _ROWS:(t + 1) * _ROWS, :] = y1.astype(jnp.bfloat16)
        s = s + jnp.sum(y1, axis=0, keepdims=True)
        q = q + jnp.sum(y1 * y1, axis=0, keepdims=True)

    @pl.when(b == 0)
    def _init():
        s1_ref[...] = s
        q1_ref[...] = q

    @pl.when(b > 0)
    def _acc():
        s1_ref[...] = s1_ref[...] + s
        q1_ref[...] = q1_ref[...] + q


def _mid_kernel(y_ref, sc_ref, sh_ref, wt_ref, bias_ref,
                o_ref, s_ref, q_ref):
    i = pl.program_id(0)
    yv = y_ref[...].astype(jnp.float32)
    zz = jnp.maximum(yv * sc_ref[...] + sh_ref[...], 0.0)
    y2 = jnp.dot(zz.astype(jnp.bfloat16), wt_ref[...],
                 preferred_element_type=jnp.float32)
    y2 = y2 + bias_ref[...]
    o_ref[...] = y2.astype(jnp.bfloat16)
    s = jnp.sum(y2, axis=0, keepdims=True)
    q = jnp.sum(y2 * y2, axis=0, keepdims=True)

    @pl.when(i == 0)
    def _init():
        s_ref[...] = s
        q_ref[...] = q

    @pl.when(i > 0)
    def _acc():
        s_ref[...] = s_ref[...] + s
        q_ref[...] = q_ref[...] + q


def _tail_kernel(x_ref, scc_ref, shc_ref, wc_ref, bsk_ref,
                 y_ref, sc_ref, sh_ref, o_ref):
    for t in range(_TB):
        xn = x_ref[t] * scc_ref[...] + shc_ref[...]
        z = jnp.zeros((_C, 1), jnp.float32)
        xm1 = jnp.concatenate([z, xn[:, :-1]], axis=1)
        xp1 = jnp.concatenate([xn[:, 1:], z], axis=1)
        xcat = jnp.concatenate([xm1, xn, xp1], axis=0)
        r = jnp.dot(wc_ref[...], xcat.astype(jnp.bfloat16),
                    preferred_element_type=jnp.float32)
        resb = jnp.maximum(r + bsk_ref[...], 0.0)
        yv = y_ref[t * _ROWS:(t + 1) * _ROWS, :].astype(jnp.float32)
        zz = jnp.maximum(yv * sc_ref[...] + sh_ref[...], 0.0)
        o_ref[t] = resb + jnp.reshape(zz, (_D2, _L))


def _finalize(s, q, gamma, beta, count):
    mu = s / count
    var = q / count - mu * mu
    sc = gamma.reshape(mu.shape) * jax.lax.rsqrt(var + _EPS)
    sh = beta.reshape(mu.shape) - mu * sc
    return sc, sh


def kernel(x, edge_index, train, gamma0, beta0, Wskip, bskip, W1, bias1,
           gamma1, beta1, W2, bias2, W3, bias3):
    del edge_index, train
    f32 = jnp.float32

    s0, q0 = pl.pallas_call(
        _xstats_kernel,
        grid=(_B // 4,),
        in_specs=[pl.BlockSpec((4, _C, _L), lambda b: (b, 0, 0))],
        out_specs=[pl.BlockSpec((_C, 1), lambda b: (0, 0)),
                   pl.BlockSpec((_C, 1), lambda b: (0, 0))],
        out_shape=[jax.ShapeDtypeStruct((_C, 1), f32),
                   jax.ShapeDtypeStruct((_C, 1), f32)],
    )(x)

    sc0, sh0 = _finalize(s0, q0, gamma0.reshape(_C, 1), beta0.reshape(_C, 1),
                         float(_B * _L))

    wc = jnp.concatenate([Wskip[:, :, 0], Wskip[:, :, 1], Wskip[:, :, 2]],
                         axis=1).astype(jnp.bfloat16)
    bsk = bskip.reshape(_D2, 1)
    w1t = W1.T.astype(jnp.bfloat16)
    b1 = bias1.reshape(1, _D2)

    y1, s1, q1 = pl.pallas_call(
        _front_kernel,
        grid=(_B // _FB,),
        in_specs=[
            pl.BlockSpec((_FB, _C, _L), lambda b: (b, 0, 0)),
            pl.BlockSpec((_C, 1), lambda b: (0, 0)),
            pl.BlockSpec((_C, 1), lambda b: (0, 0)),
            pl.BlockSpec((_C, _D2), lambda b: (0, 0)),
            pl.BlockSpec((1, _D2), lambda b: (0, 0)),
        ],
        out_specs=[
            pl.BlockSpec((_FB * _ROWS, _D2), lambda b: (b, 0)),
            pl.BlockSpec((1, _D2), lambda b: (0, 0)),
            pl.BlockSpec((1, _D2), lambda b: (0, 0)),
        ],
        out_shape=[
            jax.ShapeDtypeStruct((_N, _D2), jnp.bfloat16),
            jax.ShapeDtypeStruct((1, _D2), f32),
            jax.ShapeDtypeStruct((1, _D2), f32),
        ],
    )(x, sc0, sh0, w1t, b1)

    def mid(y, s, q, wt, bias):
        sc, sh = _finalize(s, q, gamma1.reshape(1, _D2),
                           beta1.reshape(1, _D2), float(_N))
        return pl.pallas_call(
            _mid_kernel,
            grid=(_B // _MB,),
            in_specs=[
                pl.BlockSpec((_MB * _ROWS, _D2), lambda b: (b, 0)),
                pl.BlockSpec((1, _D2), lambda b: (0, 0)),
                pl.BlockSpec((1, _D2), lambda b: (0, 0)),
                pl.BlockSpec((_D2, _D2), lambda b: (0, 0)),
                pl.BlockSpec((1, _D2), lambda b: (0, 0)),
            ],
            out_specs=[
                pl.BlockSpec((_MB * _ROWS, _D2), lambda b: (b, 0)),
                pl.BlockSpec((1, _D2), lambda b: (0, 0)),
                pl.BlockSpec((1, _D2), lambda b: (0, 0)),
            ],
            out_shape=[
                jax.ShapeDtypeStruct((_N, _D2), jnp.bfloat16),
                jax.ShapeDtypeStruct((1, _D2), f32),
                jax.ShapeDtypeStruct((1, _D2), f32),
            ],
        )(y, sc, sh, wt, bias)

    y2, s2, q2 = mid(y1, s1, q1, W2.T.astype(jnp.bfloat16),
                     bias2.reshape(1, _D2))
    y3, s3, q3 = mid(y2, s2, q2, W3.T.astype(jnp.bfloat16),
                     bias3.reshape(1, _D2))

    sc3, sh3 = _finalize(s3, q3, gamma1.reshape(1, _D2),
                         beta1.reshape(1, _D2), float(_N))
    out = pl.pallas_call(
        _tail_kernel,
        grid=(_B // _TB,),
        in_specs=[
            pl.BlockSpec((_TB, _C, _L), lambda b: (b, 0, 0)),
            pl.BlockSpec((_C, 1), lambda b: (0, 0)),
            pl.BlockSpec((_C, 1), lambda b: (0, 0)),
            pl.BlockSpec((_D2, _TK * _C), lambda b: (0, 0)),
            pl.BlockSpec((_D2, 1), lambda b: (0, 0)),
            pl.BlockSpec((_TB * _ROWS, _D2), lambda b: (b, 0)),
            pl.BlockSpec((1, _D2), lambda b: (0, 0)),
            pl.BlockSpec((1, _D2), lambda b: (0, 0)),
        ],
        out_specs=pl.BlockSpec((_TB, _D2, _L), lambda b: (b, 0, 0)),
        out_shape=jax.ShapeDtypeStruct((_B, _D2, _L), f32),
    )(x, sc0, sh0, wc, bsk, y3, sc3, sh3)

    return out


# speedup vs baseline: 2.8992x; 1.0075x over previous
import jax
import jax.numpy as jnp
from jax.experimental import pallas as pl
from jax.experimental.pallas import tpu as pltpu

_B = 16
_C = 128
_L = 2048
_D2 = 256
_TK = 3
_N = _B * _L
_ROWS = _L
_EPS = 1e-5
_SB = 4
_FB = 4
_MB = 4
_TB = 4

_CONTRACT_R1 = (((1,), (1,)), ((), ()))


def _bn0_coeffs(s_ref, q_ref, g_ref, b_ref):
    mu = s_ref[...] * (1.0 / (_B * _L))
    var = q_ref[...] * (1.0 / (_B * _L)) - mu * mu
    sc = g_ref[...] * jax.lax.rsqrt(var + _EPS)
    sh = b_ref[...] - mu * sc
    return sc, sh


def _bn_coeffs(sp_ref, qp_ref, g_ref, b_ref):
    s = jnp.sum(sp_ref[...], axis=(0, 1))[None, :]
    q = jnp.sum(qp_ref[...], axis=(0, 1))[None, :]
    mu = s * (1.0 / _N)
    var = q * (1.0 / _N) - mu * mu
    sc = g_ref[...] * jax.lax.rsqrt(var + _EPS)
    sh = b_ref[...] - mu * sc
    return sc, sh


def _xstats_kernel(x_ref, s_ref, q_ref):
    b = pl.program_id(0)
    xb = x_ref[...]
    s = jnp.sum(xb, axis=(0, 2))[:, None]
    q = jnp.sum(xb * xb, axis=(0, 2))[:, None]

    @pl.when(b == 0)
    def _init():
        s_ref[...] = s
        q_ref[...] = q

    @pl.when(b > 0)
    def _acc():
        s_ref[...] = s_ref[...] + s
        q_ref[...] = q_ref[...] + q


def _front_kernel(x_ref, s0_ref, q0_ref, g0_ref, b0_ref, w1_ref, b1_ref,
                  y1_ref, sp_ref, qp_ref):
    scc, shc = _bn0_coeffs(s0_ref, q0_ref, g0_ref, b0_ref)
    s = jnp.zeros((1, _D2), jnp.float32)
    q = jnp.zeros((1, _D2), jnp.float32)
    for t in range(_FB):
        xn = x_ref[t] * scc + shc
        hf = jnp.reshape(xn, (_ROWS, _C))
        y1 = jax.lax.dot_general(hf.astype(jnp.bfloat16), w1_ref[...],
                                 _CONTRACT_R1,
                                 preferred_element_type=jnp.float32)
        y1 = y1 + b1_ref[...]
        y1_ref[t * _ROWS:(t + 1) * _ROWS, :] = y1.astype(jnp.bfloat16)
        s = s + jnp.sum(y1, axis=0, keepdims=True)
        q = q + jnp.sum(y1 * y1, axis=0, keepdims=True)
    sp_ref[...] = s[None]
    qp_ref[...] = q[None]


def _mid_kernel(y_ref, spi_ref, qpi_ref, g_ref, b_ref, w_ref, bias_ref,
                o_ref, sp_ref, qp_ref):
    sc, sh = _bn_coeffs(spi_ref, qpi_ref, g_ref, b_ref)
    yv = y_ref[...].astype(jnp.float32)
    zz = jnp.maximum(yv * sc + sh, 0.0)
    y2 = jax.lax.dot_general(zz.astype(jnp.bfloat16), w_ref[...],
                             _CONTRACT_R1,
                             preferred_element_type=jnp.float32)
    y2 = y2 + bias_ref[...]
    o_ref[...] = y2.astype(jnp.bfloat16)
    sp_ref[...] = jnp.sum(y2, axis=0, keepdims=True)[None]
    qp_ref[...] = jnp.sum(y2 * y2, axis=0, keepdims=True)[None]


def _tail_kernel(x_ref, s0_ref, q0_ref, g0_ref, b0_ref, wc_ref, bsk_ref,
                 y_ref, spi_ref, qpi_ref, g1_ref, b1_ref, o_ref):
    scc, shc = _bn0_coeffs(s0_ref, q0_ref, g0_ref, b0_ref)
    sc, sh = _bn_coeffs(spi_ref, qpi_ref, g1_ref, b1_ref)
    for t in range(_TB):
        xn = x_ref[t] * scc + shc
        z = jnp.zeros((_C, 1), jnp.float32)
        xm1 = jnp.concatenate([z, xn[:, :-1]], axis=1)
        xp1 = jnp.concatenate([xn[:, 1:], z], axis=1)
        xcat = jnp.concatenate([xm1, xn, xp1], axis=0)
        r = jnp.dot(wc_ref[...], xcat.astype(jnp.bfloat16),
                    preferred_element_type=jnp.float32)
        resb = jnp.maximum(r + bsk_ref[...], 0.0)
        yv = y_ref[t * _ROWS:(t + 1) * _ROWS, :].astype(jnp.float32)
        zz = jnp.maximum(yv * sc + sh, 0.0)
        o_ref[t] = resb + jnp.reshape(zz, (_D2, _L))


def kernel(x, edge_index, train, gamma0, beta0, Wskip, bskip, W1, bias1,
           gamma1, beta1, W2, bias2, W3, bias3):
    del edge_index, train
    f32 = jnp.float32
    bf16 = jnp.bfloat16

    g0c = gamma0.reshape(_C, 1)
    b0c = beta0.reshape(_C, 1)
    g1r = gamma1.reshape(1, _D2)
    b1r = beta1.reshape(1, _D2)

    wc = jnp.concatenate([Wskip[:, :, 0], Wskip[:, :, 1], Wskip[:, :, 2]],
                         axis=1).astype(bf16)
    bsk = bskip.reshape(_D2, 1)

    _vec = lambda b: (0, 0)
    _vec3 = lambda b: (0, 0, 0)

    s0, q0 = pl.pallas_call(
        _xstats_kernel,
        grid=(_B // _SB,),
        in_specs=[pl.BlockSpec((_SB, _C, _L), lambda b: (b, 0, 0))],
        out_specs=[pl.BlockSpec((_C, 1), _vec),
                   pl.BlockSpec((_C, 1), _vec)],
        out_shape=[jax.ShapeDtypeStruct((_C, 1), f32),
                   jax.ShapeDtypeStruct((_C, 1), f32)],
    )(x)

    nf = _B // _FB
    y1, s1p, q1p = pl.pallas_call(
        _front_kernel,
        grid=(nf,),
        in_specs=[
            pl.BlockSpec((_FB, _C, _L), lambda b: (b, 0, 0)),
            pl.BlockSpec((_C, 1), _vec),
            pl.BlockSpec((_C, 1), _vec),
            pl.BlockSpec((_C, 1), _vec),
            pl.BlockSpec((_C, 1), _vec),
            pl.BlockSpec((_D2, _C), _vec),
            pl.BlockSpec((1, _D2), _vec),
        ],
        out_specs=[
            pl.BlockSpec((_FB * _ROWS, _D2), lambda b: (b, 0)),
            pl.BlockSpec((1, 1, _D2), lambda b: (b, 0, 0)),
            pl.BlockSpec((1, 1, _D2), lambda b: (b, 0, 0)),
        ],
        out_shape=[
            jax.ShapeDtypeStruct((_N, _D2), bf16),
            jax.ShapeDtypeStruct((nf, 1, _D2), f32),
            jax.ShapeDtypeStruct((nf, 1, _D2), f32),
        ],
    )(x, s0, q0, g0c, b0c, W1.astype(bf16), bias1.reshape(1, _D2))

    def mid(y, sp, qp, w, bias):
        nm = _B // _MB
        return pl.pallas_call(
            _mid_kernel,
            grid=(nm,),
            in_specs=[
                pl.BlockSpec((_MB * _ROWS, _D2), lambda b: (b, 0)),
                pl.BlockSpec(sp.shape, _vec3),
                pl.BlockSpec(qp.shape, _vec3),
                pl.BlockSpec((1, _D2), _vec),
                pl.BlockSpec((1, _D2), _vec),
                pl.BlockSpec((_D2, _D2), _vec),
                pl.BlockSpec((1, _D2), _vec),
            ],
            out_specs=[
                pl.BlockSpec((_MB * _ROWS, _D2), lambda b: (b, 0)),
                pl.BlockSpec((1, 1, _D2), lambda b: (b, 0, 0)),
                pl.BlockSpec((1, 1, _D2), lambda b: (b, 0, 0)),
            ],
            out_shape=[
                jax.ShapeDtypeStruct((_N, _D2), bf16),
                jax.ShapeDtypeStruct((nm, 1, _D2), f32),
                jax.ShapeDtypeStruct((nm, 1, _D2), f32),
            ],
        )(y, sp, qp, g1r, b1r, w, bias.reshape(1, _D2))

    y2, s2p, q2p = mid(y1, s1p, q1p, W2.astype(bf16), bias2)
    y3, s3p, q3p = mid(y2, s2p, q2p, W3.astype(bf16), bias3)

    out = pl.pallas_call(
        _tail_kernel,
        grid=(_B // _TB,),
        in_specs=[
            pl.BlockSpec((_TB, _C, _L), lambda b: (b, 0, 0)),
            pl.BlockSpec((_C, 1), _vec),
            pl.BlockSpec((_C, 1), _vec),
            pl.BlockSpec((_C, 1), _vec),
            pl.BlockSpec((_C, 1), _vec),
            pl.BlockSpec((_D2, _TK * _C), _vec),
            pl.BlockSpec((_D2, 1), _vec),
            pl.BlockSpec((_TB * _ROWS, _D2), lambda b: (b, 0)),
            pl.BlockSpec(s3p.shape, _vec3),
            pl.BlockSpec(q3p.shape, _vec3),
            pl.BlockSpec((1, _D2), _vec),
            pl.BlockSpec((1, _D2), _vec),
        ],
        out_specs=pl.BlockSpec((_TB, _D2, _L), lambda b: (b, 0, 0)),
        out_shape=jax.ShapeDtypeStruct((_B, _D2, _L), f32),
    )(x, s0, q0, g0c, b0c, wc, bsk, y3, s3p, q3p, g1r, b1r)

    return out


# speedup vs baseline: 3.2624x; 1.1253x over previous
import jax
import jax.numpy as jnp
from jax.experimental import pallas as pl
from jax.experimental.pallas import tpu as pltpu

_B = 16
_C = 128
_L = 2048
_D2 = 256
_TK = 3
_N = _B * _L
_ROWS = _L
_EPS = 1e-5
_SB = 4
_FB = 4
_MB = 4
_TB = 4

_CONTRACT_R1 = (((1,), (1,)), ((), ()))


def _bn0_coeffs(s_ref, q_ref, g_ref, b_ref):
    mu = s_ref[...] * (1.0 / (_B * _L))
    var = q_ref[...] * (1.0 / (_B * _L)) - mu * mu
    sc = jnp.swapaxes(g_ref[...], 0, 1) * jax.lax.rsqrt(var + _EPS)
    sh = jnp.swapaxes(b_ref[...], 0, 1) - mu * sc
    return sc, sh


def _bn_coeffs(sp_ref, qp_ref, g_ref, b_ref):
    s = jnp.sum(sp_ref[...], axis=(0, 1))[None, :]
    q = jnp.sum(qp_ref[...], axis=(0, 1))[None, :]
    mu = s * (1.0 / _N)
    var = q * (1.0 / _N) - mu * mu
    sc = g_ref[...] * jax.lax.rsqrt(var + _EPS)
    sh = b_ref[...] - mu * sc
    return sc, sh


def _xstats_kernel(x_ref, s_ref, q_ref):
    b = pl.program_id(0)
    xb = x_ref[...]
    s = jnp.sum(xb, axis=(0, 2))[:, None]
    q = jnp.sum(xb * xb, axis=(0, 2))[:, None]

    @pl.when(b == 0)
    def _init():
        s_ref[...] = s
        q_ref[...] = q

    @pl.when(b > 0)
    def _acc():
        s_ref[...] = s_ref[...] + s
        q_ref[...] = q_ref[...] + q


def _front_kernel(x_ref, s0_ref, q0_ref, g0_ref, b0_ref, w1_ref, b1_ref,
                  y1_ref, sp_ref, qp_ref):
    scc, shc = _bn0_coeffs(s0_ref, q0_ref, g0_ref, b0_ref)
    s = jnp.zeros((1, _D2), jnp.float32)
    q = jnp.zeros((1, _D2), jnp.float32)
    for t in range(_FB):
        xn = x_ref[t] * scc + shc
        hf = jnp.reshape(xn, (_ROWS, _C))
        y1 = jax.lax.dot_general(hf.astype(jnp.bfloat16),
                                 w1_ref[...].astype(jnp.bfloat16),
                                 _CONTRACT_R1,
                                 preferred_element_type=jnp.float32)
        y1 = y1 + b1_ref[...]
        y1_ref[t * _ROWS:(t + 1) * _ROWS, :] = y1.astype(jnp.bfloat16)
        s = s + jnp.sum(y1, axis=0, keepdims=True)
        q = q + jnp.sum(y1 * y1, axis=0, keepdims=True)
    sp_ref[...] = s[None]
    qp_ref[...] = q[None]


def _mid_kernel(y_ref, spi_ref, qpi_ref, g_ref, b_ref, w_ref, bias_ref,
                o_ref, sp_ref, qp_ref):
    sc, sh = _bn_coeffs(spi_ref, qpi_ref, g_ref, b_ref)
    yv = y_ref[...].astype(jnp.float32)
    zz = jnp.maximum(yv * sc + sh, 0.0)
    y2 = jax.lax.dot_general(zz.astype(jnp.bfloat16),
                             w_ref[...].astype(jnp.bfloat16),
                             _CONTRACT_R1,
                             preferred_element_type=jnp.float32)
    y2 = y2 + bias_ref[...]
    o_ref[...] = y2.astype(jnp.bfloat16)
    sp_ref[...] = jnp.sum(y2, axis=0, keepdims=True)[None]
    qp_ref[...] = jnp.sum(y2 * y2, axis=0, keepdims=True)[None]


def _tail_kernel(x_ref, s0_ref, q0_ref, g0_ref, b0_ref, wc_ref, bsk_ref,
                 y_ref, spi_ref, qpi_ref, g1_ref, b1_ref, o_ref):
    scc, shc = _bn0_coeffs(s0_ref, q0_ref, g0_ref, b0_ref)
    sc, sh = _bn_coeffs(spi_ref, qpi_ref, g1_ref, b1_ref)
    for t in range(_TB):
        xn = x_ref[t] * scc + shc
        z = jnp.zeros((_C, 1), jnp.float32)
        xm1 = jnp.concatenate([z, xn[:, :-1]], axis=1)
        xp1 = jnp.concatenate([xn[:, 1:], z], axis=1)
        xcat = jnp.concatenate([xm1, xn, xp1], axis=0)
        r = jnp.dot(wc_ref[...], xcat.astype(jnp.bfloat16),
                    preferred_element_type=jnp.float32)
        resb = jnp.maximum(r + jnp.swapaxes(bsk_ref[...], 0, 1), 0.0)
        yv = y_ref[t * _ROWS:(t + 1) * _ROWS, :].astype(jnp.float32)
        zz = jnp.maximum(yv * sc + sh, 0.0)
        o_ref[t] = resb + jnp.reshape(zz, (_D2, _L))


def kernel(x, edge_index, train, gamma0, beta0, Wskip, bskip, W1, bias1,
           gamma1, beta1, W2, bias2, W3, bias3):
    del edge_index, train
    f32 = jnp.float32
    bf16 = jnp.bfloat16

    g0c = gamma0.reshape(1, _C)
    b0c = beta0.reshape(1, _C)
    g1r = gamma1.reshape(1, _D2)
    b1r = beta1.reshape(1, _D2)

    wc = jnp.concatenate([Wskip[:, :, 0], Wskip[:, :, 1], Wskip[:, :, 2]],
                         axis=1).astype(bf16)
    bsk = bskip.reshape(1, _D2)

    _vec = lambda b: (0, 0)
    _vec3 = lambda b: (0, 0, 0)

    s0, q0 = pl.pallas_call(
        _xstats_kernel,
        grid=(_B // _SB,),
        in_specs=[pl.BlockSpec((_SB, _C, _L), lambda b: (b, 0, 0))],
        out_specs=[pl.BlockSpec((_C, 1), _vec),
                   pl.BlockSpec((_C, 1), _vec)],
        out_shape=[jax.ShapeDtypeStruct((_C, 1), f32),
                   jax.ShapeDtypeStruct((_C, 1), f32)],
    )(x)

    nf = _B // _FB
    y1, s1p, q1p = pl.pallas_call(
        _front_kernel,
        grid=(nf,),
        in_specs=[
            pl.BlockSpec((_FB, _C, _L), lambda b: (b, 0, 0)),
            pl.BlockSpec((_C, 1), _vec),
            pl.BlockSpec((_C, 1), _vec),
            pl.BlockSpec((1, _C), _vec),
            pl.BlockSpec((1, _C), _vec),
            pl.BlockSpec((_D2, _C), _vec),
            pl.BlockSpec((1, _D2), _vec),
        ],
        out_specs=[
            pl.BlockSpec((_FB * _ROWS, _D2), lambda b: (b, 0)),
            pl.BlockSpec((1, 1, _D2), lambda b: (b, 0, 0)),
            pl.BlockSpec((1, 1, _D2), lambda b: (b, 0, 0)),
        ],
        out_shape=[
            jax.ShapeDtypeStruct((_N, _D2), bf16),
            jax.ShapeDtypeStruct((nf, 1, _D2), f32),
            jax.ShapeDtypeStruct((nf, 1, _D2), f32),
        ],
    )(x, s0, q0, g0c, b0c, W1, bias1.reshape(1, _D2))

    def mid(y, sp, qp, w, bias):
        nm = _B // _MB
        return pl.pallas_call(
            _mid_kernel,
            grid=(nm,),
            in_specs=[
                pl.BlockSpec((_MB * _ROWS, _D2), lambda b: (b, 0)),
                pl.BlockSpec(sp.shape, _vec3),
                pl.BlockSpec(qp.shape, _vec3),
                pl.BlockSpec((1, _D2), _vec),
                pl.BlockSpec((1, _D2), _vec),
                pl.BlockSpec((_D2, _D2), _vec),
                pl.BlockSpec((1, _D2), _vec),
            ],
            out_specs=[
                pl.BlockSpec((_MB * _ROWS, _D2), lambda b: (b, 0)),
                pl.BlockSpec((1, 1, _D2), lambda b: (b, 0, 0)),
                pl.BlockSpec((1, 1, _D2), lambda b: (b, 0, 0)),
            ],
            out_shape=[
                jax.ShapeDtypeStruct((_N, _D2), bf16),
                jax.ShapeDtypeStruct((nm, 1, _D2), f32),
                jax.ShapeDtypeStruct((nm, 1, _D2), f32),
            ],
        )(y, sp, qp, g1r, b1r, w, bias.reshape(1, _D2))


    y2, s2p, q2p = mid(y1, s1p, q1p, W2, bias2)
    y3, s3p, q3p = mid(y2, s2p, q2p, W3, bias3)

    out = pl.pallas_call(
        _tail_kernel,
        grid=(_B // _TB,),
        in_specs=[
            pl.BlockSpec((_TB, _C, _L), lambda b: (b, 0, 0)),
            pl.BlockSpec((_C, 1), _vec),
            pl.BlockSpec((_C, 1), _vec),
            pl.BlockSpec((1, _C), _vec),
            pl.BlockSpec((1, _C), _vec),
            pl.BlockSpec((_D2, _TK * _C), _vec),
            pl.BlockSpec((1, _D2), _vec),
            pl.BlockSpec((_TB * _ROWS, _D2), lambda b: (b, 0)),
            pl.BlockSpec(s3p.shape, _vec3),
            pl.BlockSpec(q3p.shape, _vec3),
            pl.BlockSpec((1, _D2), _vec),
            pl.BlockSpec((1, _D2), _vec),
        ],
        out_specs=pl.BlockSpec((_TB, _D2, _L), lambda b: (b, 0, 0)),
        out_shape=jax.ShapeDtypeStruct((_B, _D2, _L), f32),
    )(x, s0, q0, g0c, b0c, wc, bsk, y3, s3p, q3p, g1r, b1r)

    return out
